# baseline (speedup 1.0000x reference)
import functools

import jax
import jax.numpy as jnp
from jax import lax
from jax.experimental import pallas as pl
from jax.experimental.pallas import tpu as pltpu
from jax.experimental.pallas import tpu_sc as plsc

N = 10000
E = 320000
D = 128
M = 3
ALPHA = 0.5

NC = 2
NS = 16
NW = NC * NS
NPAD = 10240
CHUNK = 128
E_PAD = 327680
EROWS = E_PAD // CHUNK
RPW = EROWS // NW
IBLK = 16
NIB = RPW // IBLK
ABLK = 40
NAB = RPW // ABLK
ZROWS = 32

BLK = 1000
GRID = N // BLK

_mesh = plsc.VectorSubcoreMesh(core_axis_name="c", subcore_axis_name="s")


def _deg_body(src_hbm, dst_hbm, out_hbm, dout_sp, din_sp, zbuf, ones_v, sidx,
              didx, osem, isem):
    core = lax.axis_index("c")
    sid = lax.axis_index("s")

    def zfill(i, _):
        zbuf[pl.ds(i * 16, 16)] = jnp.zeros((16,), jnp.float32)
        return _

    lax.fori_loop(0, 640 // 16, zfill, None)

    def ofill(i, _):
        ones_v[pl.ds(i * 16, 16)] = jnp.ones((16,), jnp.float32)
        return _

    lax.fori_loop(0, CHUNK // 16, ofill, None)

    pltpu.sync_copy(zbuf, dout_sp.at[pl.ds(sid * 640, 640)])
    pltpu.sync_copy(zbuf, din_sp.at[pl.ds(sid * 640, 640)])
    plsc.subcore_barrier()

    base_row = (core * NS + sid) * RPW

    def step(i, _):
        row = base_row + i * IBLK
        pltpu.sync_copy(src_hbm.at[pl.ds(row, IBLK)], sidx)
        pltpu.sync_copy(dst_hbm.at[pl.ds(row, IBLK)], didx)
        for r in range(IBLK):
            pltpu.async_copy(ones_v, dout_sp.at[sidx.at[r]], osem, add=True)
            pltpu.async_copy(ones_v, din_sp.at[didx.at[r]], isem, add=True)
        for r in range(IBLK):
            pltpu.make_async_copy(ones_v, dout_sp.at[sidx.at[r]], osem).wait()
            pltpu.make_async_copy(ones_v, din_sp.at[didx.at[r]], isem).wait()
        return _

    lax.fori_loop(0, NIB, step, None)
    plsc.subcore_barrier()

    pltpu.sync_copy(dout_sp.at[pl.ds(sid * 640, 640)],
                    out_hbm.at[core, 0, pl.ds(sid * 640, 640)])
    pltpu.sync_copy(din_sp.at[pl.ds(sid * 640, 640)],
                    out_hbm.at[core, 1, pl.ds(sid * 640, 640)])


_deg_call = functools.partial(
    pl.kernel,
    out_type=jax.ShapeDtypeStruct((NC, 2, NPAD), jnp.float32),
    mesh=_mesh,
    scratch_types=[
        pltpu.VMEM_SHARED((NPAD,), jnp.float32),
        pltpu.VMEM_SHARED((NPAD,), jnp.float32),
        pltpu.VMEM((640,), jnp.float32),
        pltpu.VMEM((CHUNK,), jnp.float32),
        pltpu.VMEM((IBLK, CHUNK), jnp.int32),
        pltpu.VMEM((IBLK, CHUNK), jnp.int32),
        pltpu.SemaphoreType.DMA,
        pltpu.SemaphoreType.DMA,
    ],
)(_deg_body)


def _agg_body(src_hbm, dst_hbm, hs_hbm, out_hbm, acc_sp, zb, rowbuf, sidx,
              didx, sem0, sem1, zsem):
    core = lax.axis_index("c")
    sid = lax.axis_index("s")
    base_row = (core * NS + sid) * RPW
    sems = (sem0, sem1)
    NZC = 640 // ZROWS

    def zfill(r, _):
        for cpos in range(D // 16):
            zb[r, pl.ds(cpos * 16, 16)] = jnp.zeros((16,), jnp.float32)
        return _

    lax.fori_loop(0, ZROWS, zfill, None)

    def zfire(z, _):
        pltpu.async_copy(zb, acc_sp.at[pl.ds(sid * 640 + z * ZROWS,
                                             ZROWS)], zsem)
        return _

    def zdrain(z, _):
        pltpu.make_async_copy(zb, acc_sp.at[pl.ds(sid * 640 + z * ZROWS,
                                                  ZROWS)], zsem).wait()
        return _

    lax.fori_loop(0, NZC, zfire, None)
    lax.fori_loop(0, NZC, zdrain, None)
    plsc.subcore_barrier()

    for m in range(M):
        for blk in range(NAB):
            row = base_row + blk * ABLK
            pltpu.sync_copy(src_hbm.at[pl.ds(row, ABLK)], sidx)
            pltpu.sync_copy(dst_hbm.at[pl.ds(row, ABLK)], didx)
            pltpu.async_copy(hs_hbm.at[m].at[sidx.at[0]], rowbuf.at[0], sem0)
            pltpu.async_copy(hs_hbm.at[m].at[sidx.at[1]], rowbuf.at[1], sem1)

            def chunk2(j2, _):
                for b in range(2):
                    j = j2 * 2 + b
                    pltpu.make_async_copy(hs_hbm.at[m].at[sidx.at[j]],
                                          rowbuf.at[b], sems[b]).wait()
                    pltpu.sync_copy(rowbuf.at[b], acc_sp.at[didx.at[j]],
                                    add=True)

                    @pl.when(j + 2 < ABLK)
                    def _():
                        pltpu.async_copy(hs_hbm.at[m].at[sidx.at[j + 2]],
                                         rowbuf.at[b], sems[b])
                return _

            lax.fori_loop(0, ABLK // 2, chunk2, None)
        plsc.subcore_barrier()

        pltpu.sync_copy(acc_sp.at[pl.ds(sid * 640, 640)],
                        out_hbm.at[m, core, pl.ds(sid * 640, 640)])
        if m < M - 1:
            lax.fori_loop(0, NZC, zfire, None)
            lax.fori_loop(0, NZC, zdrain, None)
        plsc.subcore_barrier()


_agg_call = functools.partial(
    pl.kernel,
    out_type=jax.ShapeDtypeStruct((M, NC, NPAD, D), jnp.float32),
    mesh=_mesh,
    scratch_types=[
        pltpu.VMEM_SHARED((NPAD, D), jnp.float32),
        pltpu.VMEM((ZROWS, D), jnp.float32),
        pltpu.VMEM((2, CHUNK, D), jnp.float32),
        pltpu.VMEM((ABLK, CHUNK), jnp.int32),
        pltpu.VMEM((ABLK, CHUNK), jnp.int32),
        pltpu.SemaphoreType.DMA,
        pltpu.SemaphoreType.DMA,
        pltpu.SemaphoreType.DMA,
    ],
)(_agg_body)


def _proj_body(f0, f1, f2, wp, bp, wa, proj, sacc):
    i = pl.program_id(0)
    feats = (f0[...], f1[...], f2[...])
    tns = []
    for m in range(M):
        p = lax.dot_general(feats[m], wp[m], (((1,), (1,)), ((), ())),
                            preferred_element_type=jnp.float32)
        p = jnp.maximum(p + bp[m:m + 1, :], 0.0)
        proj[m] = p
        t = lax.dot_general(p, wa[m], (((1,), (1,)), ((), ())),
                            preferred_element_type=jnp.float32)
        nrm = jnp.maximum(jnp.sqrt(jnp.sum(t * t, axis=1, keepdims=True)),
                          1e-12)
        tns.append(t / nrm)
    pairs = ((0, 0), (0, 1), (0, 2), (1, 1), (1, 2), (2, 2))
    rows = [jnp.full((1, 128), jnp.sum(tns[a] * tns[b]), jnp.float32)
            for a, b in pairs]
    tile = jnp.concatenate(rows + [jnp.zeros((2, 128), jnp.float32)], axis=0)

    @pl.when(i == 0)
    def _():
        sacc[...] = tile

    @pl.when(i != 0)
    def _():
        sacc[...] = sacc[...] + tile


def _proj_call(f0, f1, f2, wp, bp, wa):
    return pl.pallas_call(
        _proj_body,
        grid=(GRID,),
        in_specs=[
            pl.BlockSpec((BLK, D), lambda i: (i, 0)),
            pl.BlockSpec((BLK, D), lambda i: (i, 0)),
            pl.BlockSpec((BLK, D), lambda i: (i, 0)),
            pl.BlockSpec((M, D, D), lambda i: (0, 0, 0)),
            pl.BlockSpec((M, D), lambda i: (0, 0)),
            pl.BlockSpec((M, D, D), lambda i: (0, 0, 0)),
        ],
        out_specs=[
            pl.BlockSpec((M, BLK, D), lambda i: (0, i, 0)),
            pl.BlockSpec((8, 128), lambda i: (0, 0)),
        ],
        out_shape=[
            jax.ShapeDtypeStruct((M, N, D), jnp.float32),
            jax.ShapeDtypeStruct((8, 128), jnp.float32),
        ],
    )(f0, f1, f2, wp, bp, wa)


def _scale_body(proj, dout, hs):
    deg = dout[:, 0:1] + dout[:, 1:2]
    nsrc = lax.rsqrt(jnp.maximum(deg, 1.0))
    for m in range(M):
        hs[m] = proj[m] * nsrc


def _scale_call(proj, dout_t):
    return pl.pallas_call(
        _scale_body,
        grid=(GRID,),
        in_specs=[
            pl.BlockSpec((M, BLK, D), lambda i: (0, i, 0)),
            pl.BlockSpec((BLK, 2), lambda i: (i, 0)),
        ],
        out_specs=pl.BlockSpec((M, BLK, D), lambda i: (0, i, 0)),
        out_shape=jax.ShapeDtypeStruct((M, NPAD, D), jnp.float32),
    )(proj, dout_t)


def _final_body(sacc, aggp, din, proj, wg, bg, wlt, blt, o0, o1, o2):
    deg = din[:, 0:1] + din[:, 1:2]
    ndst = lax.rsqrt(jnp.maximum(deg, 1.0))
    s = sacc[...] * (1.0 / N)
    sv = [s[k:k + 1, :] for k in range(6)]
    smat = ((sv[0], sv[1], sv[2]), (sv[1], sv[3], sv[4]), (sv[2], sv[4], sv[5]))
    wrows = []
    for d in range(M):
        r = smat[d]
        mx = jnp.maximum(jnp.maximum(r[0], r[1]), r[2])
        es = [jnp.exp(r[g] - mx) for g in range(M)]
        tot = es[0] + es[1] + es[2]
        wrows.append([e / tot for e in es])
    ps = [proj[m] for m in range(M)]
    outs = (o0, o1, o2)
    for m in range(M):
        agg = (aggp[m, 0] + aggp[m, 1]) * ndst
        intra = lax.dot_general(agg, wg[m], (((1,), (0,)), ((), ())),
                                preferred_element_type=jnp.float32)
        intra = jnp.maximum(intra + bg[m:m + 1, :], 0.0)
        mix = wrows[m][0] * ps[0] + wrows[m][1] * ps[1] + wrows[m][2] * ps[2]
        inter = lax.dot_general(mix, wlt[...], (((1,), (1,)), ((), ())),
                                preferred_element_type=jnp.float32) + blt[...]
        outs[m][...] = ALPHA * intra + (1.0 - ALPHA) * inter


def _final_call(sacc, aggp, din_t, proj, wg, bg, wlt, blt2d):
    return pl.pallas_call(
        _final_body,
        grid=(GRID,),
        in_specs=[
            pl.BlockSpec((8, 128), lambda i: (0, 0)),
            pl.BlockSpec((M, NC, BLK, D), lambda i: (0, 0, i, 0)),
            pl.BlockSpec((BLK, 2), lambda i: (i, 0)),
            pl.BlockSpec((M, BLK, D), lambda i: (0, i, 0)),
            pl.BlockSpec((M, D, D), lambda i: (0, 0, 0)),
            pl.BlockSpec((M, D), lambda i: (0, 0)),
            pl.BlockSpec((D, D), lambda i: (0, 0)),
            pl.BlockSpec((1, D), lambda i: (0, 0)),
        ],
        out_specs=[
            pl.BlockSpec((BLK, D), lambda i: (i, 0)),
            pl.BlockSpec((BLK, D), lambda i: (i, 0)),
            pl.BlockSpec((BLK, D), lambda i: (i, 0)),
        ],
        out_shape=[
            jax.ShapeDtypeStruct((N, D), jnp.float32),
            jax.ShapeDtypeStruct((N, D), jnp.float32),
            jax.ShapeDtypeStruct((N, D), jnp.float32),
        ],
    )(sacc, aggp, din_t, proj, wg, bg, wlt, blt2d)


def kernel(feat0, feat1, feat2, edge_index, Wp, bp, Wg, bg, Wa, Wlt, blt):
    pad = N + (jnp.arange(E_PAD - E, dtype=jnp.int32) % (NPAD - N))
    src2d = jnp.concatenate([edge_index[0].astype(jnp.int32), pad]
                            ).reshape(EROWS, CHUNK)
    dst2d = jnp.concatenate([edge_index[1].astype(jnp.int32), pad]
                            ).reshape(EROWS, CHUNK)

    degs = _deg_call(src2d, dst2d)
    dout_t = jnp.transpose(degs[:, 0, :N])
    din_t = jnp.transpose(degs[:, 1, :N])

    proj, sacc = _proj_call(feat0, feat1, feat2, Wp, bp, Wa)
    hs = _scale_call(proj, dout_t)

    aggp = _agg_call(src2d, dst2d, hs)

    o0, o1, o2 = _final_call(sacc, aggp, din_t, proj, Wg, bg, Wlt,
                             blt.reshape(1, D))
    return (o0, o1, o2)
